# baseline (speedup 1.0000x reference)
import functools

import jax
import jax.numpy as jnp
from jax import lax
from jax.experimental import pallas as pl
from jax.experimental.pallas import tpu as pltpu
from jax.experimental.pallas import tpu_sc as plsc

B = 16384
D_MF = 8
D_MLP = 32
N_ROWS = 1_000_000
G_MF = 128 // D_MF
G_MLP = 128 // D_MLP
NC = 2
NS = 16
NW = NC * NS
BPW = B // NW

BLK = 2048
GRID = B // BLK

_sc_mesh = plsc.VectorSubcoreMesh(core_axis_name="c", subcore_axis_name="s")


@functools.partial(
    pl.kernel,
    mesh=_sc_mesh,
    compiler_params=pltpu.CompilerParams(use_tc_tiling_on_sc=True),
    out_type=[
        jax.ShapeDtypeStruct((B, 128), jnp.float32),
        jax.ShapeDtypeStruct((B, 128), jnp.float32),
        jax.ShapeDtypeStruct((B, 128), jnp.float32),
        jax.ShapeDtypeStruct((B, 128), jnp.float32),
    ],
    scratch_types=[
        pltpu.VMEM((BPW,), jnp.int32),
        pltpu.VMEM((BPW, 128), jnp.float32),
        pltpu.SemaphoreType.DMA,
    ],
)
def _sc_gather(gumf_hbm, gimf_hbm, gumlp_hbm, gimlp_hbm,
               umf_hbm, imf_hbm, umlp_hbm, imlp_hbm,
               umf_out, imf_out, umlp_out, imlp_out,
               idx_v, buf_v, sem):
    wid = lax.axis_index("s") * NC + lax.axis_index("c")
    base = wid * BPW
    for g_hbm, tbl_hbm, out_hbm in (
        (gumf_hbm, umf_hbm, umf_out),
        (gimf_hbm, imf_hbm, imf_out),
        (gumlp_hbm, umlp_hbm, umlp_out),
        (gimlp_hbm, imlp_hbm, imlp_out),
    ):
        pltpu.sync_copy(g_hbm.at[pl.ds(base, BPW)], idx_v)
        pltpu.async_copy(tbl_hbm.at[idx_v], buf_v, sem).wait()
        pltpu.sync_copy(buf_v, out_hbm.at[pl.ds(base, BPW)])


def _roll_left(x, s):
    return jnp.concatenate([x[:, s:], x[:, :s]], axis=1)


def _extract(x, off, width):
    s = width
    while s < 128:
        x = jnp.where((off & s) != 0, _roll_left(x, s), x)
        s *= 2
    return x[:, :width]


def _tc_mlp_body(gumf, gimf, gumlp, gimlp, o_mf_u, o_mf_i, o_mlp_u, o_mlp_i,
                 w1a, w1b, b1, w2, b2, wo_mf, wo_h, bo, out):
    ue_mf = _extract(gumf[...], o_mf_u[...], D_MF)
    ie_mf = _extract(gimf[...], o_mf_i[...], D_MF)
    ue_mlp = _extract(gumlp[...], o_mlp_u[...], D_MLP)
    ie_mlp = _extract(gimlp[...], o_mlp_i[...], D_MLP)
    h = jnp.dot(ue_mlp, w1a[...], preferred_element_type=jnp.float32)
    h = h + jnp.dot(ie_mlp, w1b[...], preferred_element_type=jnp.float32)
    h = jnp.maximum(h + b1[...], 0.0)
    h = jnp.dot(h, w2[...], preferred_element_type=jnp.float32) + b2[...]
    h = jnp.maximum(h, 0.0)
    mf = ue_mf * ie_mf
    o = jnp.dot(mf, wo_mf[...], preferred_element_type=jnp.float32)
    o = o + jnp.dot(h, wo_h[...], preferred_element_type=jnp.float32)
    out[...] = o + bo[...]


def _tc_mlp(gumf, gimf, gumlp, gimlp, o_mf_u, o_mf_i, o_mlp_u, o_mlp_i,
            W1, b1, W2, b2, Wo, bo):
    w1a = W1[:D_MLP]
    w1b = W1[D_MLP:]
    wo_mf = Wo[:D_MF]
    wo_h = Wo[D_MF:]
    blk = lambda i: (i, 0)
    full = lambda i: (0, 0)
    return pl.pallas_call(
        _tc_mlp_body,
        grid=(GRID,),
        in_specs=[
            pl.BlockSpec((BLK, 128), blk),
            pl.BlockSpec((BLK, 128), blk),
            pl.BlockSpec((BLK, 128), blk),
            pl.BlockSpec((BLK, 128), blk),
            pl.BlockSpec((BLK, 1), blk),
            pl.BlockSpec((BLK, 1), blk),
            pl.BlockSpec((BLK, 1), blk),
            pl.BlockSpec((BLK, 1), blk),
            pl.BlockSpec(w1a.shape, full),
            pl.BlockSpec(w1b.shape, full),
            pl.BlockSpec((1, D_MLP), full),
            pl.BlockSpec(W2.shape, full),
            pl.BlockSpec((1, 16), full),
            pl.BlockSpec(wo_mf.shape, full),
            pl.BlockSpec(wo_h.shape, full),
            pl.BlockSpec((1, 1), full),
        ],
        out_specs=pl.BlockSpec((BLK, 1), blk),
        out_shape=jax.ShapeDtypeStruct((B, 1), jnp.float32),
    )(gumf, gimf, gumlp, gimlp, o_mf_u, o_mf_i, o_mlp_u, o_mlp_i,
      w1a, w1b, b1.reshape(1, -1), W2, b2.reshape(1, -1),
      wo_mf, wo_h, bo.reshape(1, 1))


def kernel(user_ids, item_ids, user_mf, item_mf, user_mlp, item_mlp,
           W1, b1, W2, b2, Wo, bo):
    uid = user_ids.astype(jnp.int32)
    iid = item_ids.astype(jnp.int32)
    g_umf = uid >> 4
    g_imf = iid >> 4
    g_umlp = uid >> 2
    g_imlp = iid >> 2
    umf128 = user_mf.reshape(N_ROWS // G_MF, 128)
    imf128 = item_mf.reshape(N_ROWS // G_MF, 128)
    umlp128 = user_mlp.reshape(N_ROWS // G_MLP, 128)
    imlp128 = item_mlp.reshape(N_ROWS // G_MLP, 128)
    gumf, gimf, gumlp, gimlp = _sc_gather(
        g_umf, g_imf, g_umlp, g_imlp, umf128, imf128, umlp128, imlp128)
    o_mf_u = ((uid & (G_MF - 1)) << 3).reshape(B, 1)
    o_mf_i = ((iid & (G_MF - 1)) << 3).reshape(B, 1)
    o_mlp_u = ((uid & (G_MLP - 1)) << 5).reshape(B, 1)
    o_mlp_i = ((iid & (G_MLP - 1)) << 5).reshape(B, 1)
    out = _tc_mlp(gumf, gimf, gumlp, gimlp, o_mf_u, o_mf_i, o_mlp_u, o_mlp_i,
                  W1, b1, W2, b2, Wo, bo)
    return out[:, 0]


# speedup vs baseline: 1.0006x; 1.0006x over previous
import functools

import jax
import jax.numpy as jnp
from jax import lax
from jax.experimental import pallas as pl
from jax.experimental.pallas import tpu as pltpu
from jax.experimental.pallas import tpu_sc as plsc

B = 16384
D_MF = 8
D_MLP = 32
N_ROWS = 1_000_000
G_MF = 128 // D_MF
G_MLP = 128 // D_MLP
NC = 2
NS = 16
NW = NC * NS
BPW = B // NW

BLK = 2048
GRID = B // BLK

_sc_mesh = plsc.VectorSubcoreMesh(core_axis_name="c", subcore_axis_name="s")


@functools.partial(
    pl.kernel,
    mesh=_sc_mesh,
    out_type=[
        jax.ShapeDtypeStruct((B, 128), jnp.float32),
        jax.ShapeDtypeStruct((B, 128), jnp.float32),
        jax.ShapeDtypeStruct((B, 128), jnp.float32),
        jax.ShapeDtypeStruct((B, 128), jnp.float32),
    ],
    scratch_types=[
        pltpu.VMEM((BPW,), jnp.int32),
        pltpu.VMEM((BPW, 128), jnp.float32),
        pltpu.SemaphoreType.DMA,
    ],
)
def _sc_gather(gumf_hbm, gimf_hbm, gumlp_hbm, gimlp_hbm,
               umf_hbm, imf_hbm, umlp_hbm, imlp_hbm,
               umf_out, imf_out, umlp_out, imlp_out,
               idx_v, buf_v, sem):
    wid = lax.axis_index("s") * NC + lax.axis_index("c")
    base = wid * BPW
    for g_hbm, tbl_hbm, out_hbm in (
        (gumf_hbm, umf_hbm, umf_out),
        (gimf_hbm, imf_hbm, imf_out),
        (gumlp_hbm, umlp_hbm, umlp_out),
        (gimlp_hbm, imlp_hbm, imlp_out),
    ):
        pltpu.sync_copy(g_hbm.at[pl.ds(base, BPW)], idx_v)
        pltpu.async_copy(tbl_hbm.at[idx_v], buf_v, sem).wait()
        pltpu.sync_copy(buf_v, out_hbm.at[pl.ds(base, BPW)])


def _roll_left(x, s):
    return jnp.concatenate([x[:, s:], x[:, :s]], axis=1)


def _extract(x, off, width):
    s = width
    while s < 128:
        x = jnp.where((off & s) != 0, _roll_left(x, s), x)
        s *= 2
    return x[:, :width]


def _tc_mlp_body(gumf, gimf, gumlp, gimlp, o_mf_u, o_mf_i, o_mlp_u, o_mlp_i,
                 w1a, w1b, b1, w2, b2, wo_mf, wo_h, bo, out):
    ue_mf = _extract(gumf[...], o_mf_u[...], D_MF)
    ie_mf = _extract(gimf[...], o_mf_i[...], D_MF)
    ue_mlp = _extract(gumlp[...], o_mlp_u[...], D_MLP)
    ie_mlp = _extract(gimlp[...], o_mlp_i[...], D_MLP)
    h = jnp.dot(ue_mlp, w1a[...], preferred_element_type=jnp.float32)
    h = h + jnp.dot(ie_mlp, w1b[...], preferred_element_type=jnp.float32)
    h = jnp.maximum(h + b1[...], 0.0)
    h = jnp.dot(h, w2[...], preferred_element_type=jnp.float32) + b2[...]
    h = jnp.maximum(h, 0.0)
    mf = ue_mf * ie_mf
    o = jnp.dot(mf, wo_mf[...], preferred_element_type=jnp.float32)
    o = o + jnp.dot(h, wo_h[...], preferred_element_type=jnp.float32)
    out[...] = o + bo[...]


def _tc_mlp(gumf, gimf, gumlp, gimlp, o_mf_u, o_mf_i, o_mlp_u, o_mlp_i,
            W1, b1, W2, b2, Wo, bo):
    w1a = W1[:D_MLP]
    w1b = W1[D_MLP:]
    wo_mf = Wo[:D_MF]
    wo_h = Wo[D_MF:]
    blk = lambda i: (i, 0)
    full = lambda i: (0, 0)
    return pl.pallas_call(
        _tc_mlp_body,
        grid=(GRID,),
        in_specs=[
            pl.BlockSpec((BLK, 128), blk),
            pl.BlockSpec((BLK, 128), blk),
            pl.BlockSpec((BLK, 128), blk),
            pl.BlockSpec((BLK, 128), blk),
            pl.BlockSpec((BLK, 1), blk),
            pl.BlockSpec((BLK, 1), blk),
            pl.BlockSpec((BLK, 1), blk),
            pl.BlockSpec((BLK, 1), blk),
            pl.BlockSpec(w1a.shape, full),
            pl.BlockSpec(w1b.shape, full),
            pl.BlockSpec((1, D_MLP), full),
            pl.BlockSpec(W2.shape, full),
            pl.BlockSpec((1, 16), full),
            pl.BlockSpec(wo_mf.shape, full),
            pl.BlockSpec(wo_h.shape, full),
            pl.BlockSpec((1, 1), full),
        ],
        out_specs=pl.BlockSpec((BLK, 1), blk),
        out_shape=jax.ShapeDtypeStruct((B, 1), jnp.float32),
    )(gumf, gimf, gumlp, gimlp, o_mf_u, o_mf_i, o_mlp_u, o_mlp_i,
      w1a, w1b, b1.reshape(1, -1), W2, b2.reshape(1, -1),
      wo_mf, wo_h, bo.reshape(1, 1))


def kernel(user_ids, item_ids, user_mf, item_mf, user_mlp, item_mlp,
           W1, b1, W2, b2, Wo, bo):
    uid = user_ids.astype(jnp.int32)
    iid = item_ids.astype(jnp.int32)
    g_umf = uid >> 4
    g_imf = iid >> 4
    g_umlp = uid >> 2
    g_imlp = iid >> 2
    umf128 = user_mf.reshape(N_ROWS // G_MF, 128)
    imf128 = item_mf.reshape(N_ROWS // G_MF, 128)
    umlp128 = user_mlp.reshape(N_ROWS // G_MLP, 128)
    imlp128 = item_mlp.reshape(N_ROWS // G_MLP, 128)
    gumf, gimf, gumlp, gimlp = _sc_gather(
        g_umf, g_imf, g_umlp, g_imlp, umf128, imf128, umlp128, imlp128)
    o_mf_u = ((uid & (G_MF - 1)) << 3).reshape(B, 1)
    o_mf_i = ((iid & (G_MF - 1)) << 3).reshape(B, 1)
    o_mlp_u = ((uid & (G_MLP - 1)) << 5).reshape(B, 1)
    o_mlp_i = ((iid & (G_MLP - 1)) << 5).reshape(B, 1)
    out = _tc_mlp(gumf, gimf, gumlp, gimlp, o_mf_u, o_mf_i, o_mlp_u, o_mlp_i,
                  W1, b1, W2, b2, Wo, bo)
    return out[:, 0]
